# baseline (speedup 1.0000x reference)
import functools

import jax
import jax.numpy as jnp
from jax import lax
from jax.experimental import pallas as pl
from jax.experimental.pallas import tpu as pltpu
from jax.experimental.pallas import tpu_sc as plsc

_H = 768
_F = 1536
_E = 8
_T = 2048
_TT = 512
_NT = _T // _TT
_G = 128
_M = 2 * _T + _E * _G
_NR = _M // _G
_NC = 2
_NS = 16
_NW = _NC * _NS
_PAIRS = 2 * _T
_PPW = _PAIRS // _NW
_TPW = _T // _NW


def _route_body(x_ref, rw_ref, logits_ref, d1_ref, d2_ref, te_ref, re_ref,
                nxt_ref, nxt2_ref,
                logits_s, rank_s, idx1_s, idx2_s, cnt_s, tri_s):
    i = pl.program_id(0)

    @pl.when(i < _NT)
    def _():
        sl = pl.ds(i * _TT, _TT)

        @pl.when(i == 0)
        def _():
            cnt_s[...] = jnp.zeros_like(cnt_s)
            ri = lax.broadcasted_iota(jnp.int32, (_TT, _TT), 0)
            ci = lax.broadcasted_iota(jnp.int32, (_TT, _TT), 1)
            tri_s[...] = (ci < ri).astype(jnp.float32)

        x = x_ref[...]
        logits = lax.dot_general(
            x, rw_ref[...], (((1,), (0,)), ((), ())),
            precision=lax.Precision.DEFAULT,
            preferred_element_type=jnp.float32,
        )
        logits_s[sl, :] = logits
        lane = lax.broadcasted_iota(jnp.int32, (_TT, _E), 1)
        m1 = jnp.max(logits, axis=1, keepdims=True)
        idx1 = jnp.min(jnp.where(logits == m1, lane, _E), axis=1,
                       keepdims=True)
        neg = jnp.full_like(logits, -jnp.inf)
        l2 = jnp.where(lane == idx1, neg, logits)
        m2 = jnp.max(l2, axis=1, keepdims=True)
        idx2 = jnp.min(jnp.where(l2 == m2, lane, _E), axis=1, keepdims=True)
        idx1_s[sl, :] = idx1
        idx2_s[sl, :] = idx2
        mask = ((lane == idx1) | (lane == idx2)).astype(jnp.float32)
        rank = lax.dot_general(
            tri_s[...], mask, (((1,), (0,)), ((), ())),
            precision=lax.Precision.DEFAULT,
            preferred_element_type=jnp.float32,
        )
        rank_s[sl, :] = rank + cnt_s[...]
        cnt_s[...] = cnt_s[...] + jnp.sum(mask, axis=0, keepdims=True)

    @pl.when(i == _NT)
    def _():
        logits_ref[...] = logits_s[...]
        counts = cnt_s[...]
        pc = jnp.floor((counts + (_G - 1)) * (1.0 / _G)) * _G
        er = lax.broadcasted_iota(jnp.int32, (_E, _E), 0)
        ec = lax.broadcasted_iota(jnp.int32, (_E, _E), 1)
        ltri = (er < ec).astype(jnp.float32)
        poff = lax.dot_general(
            pc, ltri, (((1,), (0,)), ((), ())),
            precision=lax.Precision.DEFAULT,
            preferred_element_type=jnp.float32,
        )
        rank = rank_s[...]
        idx1 = idx1_s[...]
        idx2 = idx2_s[...]
        lane = lax.broadcasted_iota(jnp.int32, (_T, _E), 1)
        val = poff + rank
        d1 = jnp.sum(jnp.where(lane == idx1, val, 0.0), axis=1, keepdims=True)
        d2 = jnp.sum(jnp.where(lane == idx2, val, 0.0), axis=1, keepdims=True)
        d1_ref[...] = d1.astype(jnp.int32)
        d2_ref[...] = d2.astype(jnp.int32)
        ends = poff + pc
        rows = (lax.broadcasted_iota(jnp.int32, (_NR, _E), 0) * _G
                ).astype(jnp.float32)
        te = jnp.sum((rows >= ends).astype(jnp.int32), axis=1, keepdims=True)
        lane_g = lax.broadcasted_iota(jnp.int32, (1, _E), 1)
        gmax = jnp.max(jnp.where(pc > 0.0, lane_g, 0), axis=1, keepdims=True)
        te = jnp.minimum(te, gmax)
        te_ref[...] = te
        re_ref[...] = jnp.sum(pc, axis=1, keepdims=True).astype(jnp.int32)
        lane_r = lax.broadcasted_iota(jnp.int32, (_NR, _E), 1)
        present = pc > 0.0
        cand = jnp.where((lane_r > te) & present, lane_r, _E)
        nxt = jnp.min(cand, axis=1, keepdims=True)
        nxt = jnp.where(nxt == _E, te, nxt)
        nxt_ref[...] = nxt
        cand2 = jnp.where((lane_r > nxt) & present, lane_r, _E)
        nxt2 = jnp.min(cand2, axis=1, keepdims=True)
        nxt2_ref[...] = jnp.where(nxt2 == _E, nxt, nxt2)


_DCH = 4


def _dispatch_body(flat_hbm, dest_hbm, xs_hbm, dest_v, rbufs, gsems, ssems):
    c = lax.axis_index("c")
    s = lax.axis_index("s")
    wid = s * _NC + c
    ck = _PPW // _DCH
    pltpu.sync_copy(dest_hbm.at[wid], dest_v)
    tbase = lax.rem(wid, _NS) * _PPW
    gets = []
    for ch in range(_DCH):
        gets.append(pltpu.async_copy(
            flat_hbm.at[pl.ds(tbase + ch * ck, ck)], rbufs.at[ch],
            gsems.at[ch]))
    puts = []
    for ch in range(_DCH):
        gets[ch].wait()
        puts.append(pltpu.async_copy(
            rbufs.at[ch], xs_hbm.at[dest_v.at[ch]], ssems.at[ch]))
    for ch in range(_DCH):
        puts[ch].wait()


def _gmm_body(te_ref, re_ref, nx_ref, nx2_ref, xs_ref, upw_hbm, upb_ref,
              dww_hbm, dwb_ref,
              ys_ref, upw_b, dww_b, sems, ns_ref):
    r = pl.program_id(0)
    e = te_ref[r]
    prev_e = jnp.where(r > 0, te_ref[jnp.maximum(r - 1, 0)], -1)
    switch = e != prev_e
    ne = nx_ref[r]

    ne2 = nx2_ref[r]

    @pl.when(r == 0)
    def _():
        ns_ref[0] = 0
        ns_ref[1] = 0
        pltpu.make_async_copy(upw_hbm.at[e], upw_b.at[0],
                              sems.at[0, 0]).start()
        pltpu.make_async_copy(dww_hbm.at[e], dww_b.at[0],
                              sems.at[0, 1]).start()
        ns_ref[0] = 1

        @pl.when(ne != e)
        def _():
            pltpu.make_async_copy(upw_hbm.at[ne], upw_b.at[1],
                                  sems.at[1, 0]).start()
            pltpu.make_async_copy(dww_hbm.at[ne], dww_b.at[1],
                                  sems.at[1, 1]).start()
            ns_ref[0] = 2

    @pl.when(switch)
    def _():
        wslot = lax.rem(ns_ref[1], 3)
        for _k in range(3):
            @pl.when(wslot == _k)
            def _(_k=_k):
                pltpu.make_async_copy(upw_hbm.at[e], upw_b.at[_k],
                                      sems.at[_k, 0]).wait()
                pltpu.make_async_copy(dww_hbm.at[e], dww_b.at[_k],
                                      sems.at[_k, 1]).wait()
        ns_ref[1] = ns_ref[1] + 1

        @pl.when(ne2 != ne)
        def _():
            islot = lax.rem(ns_ref[0], 3)
            for _k in range(3):
                @pl.when(islot == _k)
                def _(_k=_k):
                    pltpu.make_async_copy(upw_hbm.at[ne2], upw_b.at[_k],
                                          sems.at[_k, 0]).start()
                    pltpu.make_async_copy(dww_hbm.at[ne2], dww_b.at[_k],
                                          sems.at[_k, 1]).start()
            ns_ref[0] = ns_ref[0] + 1

    slot = lax.rem(ns_ref[1] - 1, 3)
    active = r * _G < re_ref[0]

    def _compute(k):
        up = lax.dot_general(
            xs_ref[...], upw_b[k], (((1,), (0,)), ((), ())),
            precision=lax.Precision.DEFAULT,
            preferred_element_type=jnp.float32,
        )
        up = up + upb_ref[0]
        act = up * jax.nn.sigmoid(up)
        down = lax.dot_general(
            act, dww_b[k], (((1,), (0,)), ((), ())),
            precision=lax.Precision.DEFAULT,
            preferred_element_type=jnp.float32,
        )
        ys_ref[...] = down + dwb_ref[0]

    for _k in range(3):
        @pl.when(active & (slot == _k))
        def _(_k=_k):
            _compute(_k)


_CC = 32


def _combine_body(ys_hbm, d1_hbm, d2_hbm, out_hbm, i1, i2, r1, r2,
                  sem1, sem2, semw):
    c = lax.axis_index("c")
    s = lax.axis_index("s")
    wid = s * _NC + c
    nch = _TPW // _CC
    pltpu.sync_copy(d1_hbm.at[wid], i1)
    pltpu.sync_copy(d2_hbm.at[wid], i2)
    g1s, g2s = [], []
    for ch in range(nch):
        g1s.append(pltpu.async_copy(ys_hbm.at[i1.at[ch]], r1.at[ch],
                                    sem1.at[ch]))
        g2s.append(pltpu.async_copy(ys_hbm.at[i2.at[ch]], r2.at[ch],
                                    sem2.at[ch]))
    puts = []
    for ch in range(nch):
        g1s[ch].wait()
        g2s[ch].wait()

        def _row(row, carry, ch=ch):
            for col in range(_H // 16):
                sl = (ch, row, pl.ds(col * 16, 16))
                r1[sl] = r1[sl] + r2[sl]
            return carry

        lax.fori_loop(0, _CC, _row, 0)
        puts.append(pltpu.async_copy(
            r1.at[ch], out_hbm.at[pl.ds(wid * _TPW + ch * _CC, _CC)],
            semw.at[ch]))
    for p in puts:
        p.wait()


@functools.partial(jax.jit, static_argnames=())
def kernel(hidden_states, router_w, up_w, up_b, down_w, down_b):
    b, s_len, d = hidden_states.shape
    flat = hidden_states.reshape(_T, d)

    logits, d1, d2, te, rowend, nxt, nxt2 = pl.pallas_call(
        _route_body,
        grid=(_NT + 1,),
        in_specs=[
            pl.BlockSpec((_TT, _H), lambda i: (jnp.minimum(i, _NT - 1), 0)),
            pl.BlockSpec((_H, _E), lambda i: (0, 0)),
        ],
        out_specs=(
            pl.BlockSpec((_T, _E), lambda i: (0, 0)),
            pl.BlockSpec((_T, 1), lambda i: (0, 0)),
            pl.BlockSpec((_T, 1), lambda i: (0, 0)),
            pl.BlockSpec((_NR, 1), lambda i: (0, 0)),
            pl.BlockSpec((1, 1), lambda i: (0, 0)),
            pl.BlockSpec((_NR, 1), lambda i: (0, 0)),
            pl.BlockSpec((_NR, 1), lambda i: (0, 0)),
        ),
        out_shape=(
            jax.ShapeDtypeStruct((_T, _E), jnp.float32),
            jax.ShapeDtypeStruct((_T, 1), jnp.int32),
            jax.ShapeDtypeStruct((_T, 1), jnp.int32),
            jax.ShapeDtypeStruct((_NR, 1), jnp.int32),
            jax.ShapeDtypeStruct((1, 1), jnp.int32),
            jax.ShapeDtypeStruct((_NR, 1), jnp.int32),
            jax.ShapeDtypeStruct((_NR, 1), jnp.int32),
        ),
        scratch_shapes=[
            pltpu.VMEM((_T, _E), jnp.float32),
            pltpu.VMEM((_T, _E), jnp.float32),
            pltpu.VMEM((_T, 1), jnp.int32),
            pltpu.VMEM((_T, 1), jnp.int32),
            pltpu.VMEM((1, _E), jnp.float32),
            pltpu.VMEM((_TT, _TT), jnp.float32),
        ],
        compiler_params=pltpu.CompilerParams(
            dimension_semantics=("arbitrary",),
        ),
    )(flat, router_w)

    dest = jnp.concatenate([d1.reshape(_T), d2.reshape(_T)])
    dest3 = dest.reshape(_NW, _DCH, _PPW // _DCH)

    mesh = plsc.VectorSubcoreMesh(core_axis_name="c", subcore_axis_name="s")
    xs = pl.kernel(
        _dispatch_body,
        out_type=jax.ShapeDtypeStruct((_M, _H), jnp.float32),
        mesh=mesh,
        scratch_types=[
            pltpu.VMEM((_DCH, _PPW // _DCH), jnp.int32),
            pltpu.VMEM((_DCH, _PPW // _DCH, _H), jnp.float32),
            pltpu.SemaphoreType.DMA((_DCH,)),
            pltpu.SemaphoreType.DMA((_DCH,)),
        ],
    )(flat, dest3)

    up_b3 = up_b.reshape(_E, 1, _F)
    down_b3 = down_b.reshape(_E, 1, _H)
    ys = pl.pallas_call(
        _gmm_body,
        grid_spec=pltpu.PrefetchScalarGridSpec(
            num_scalar_prefetch=4,
            grid=(_NR,),
            in_specs=[
                pl.BlockSpec((_G, _H), lambda r, te_s, re_s, nx_s, nx2_s: (r, 0)),
                pl.BlockSpec(memory_space=pl.ANY),
                pl.BlockSpec((1, 1, _F),
                             lambda r, te_s, re_s, nx_s, nx2_s: (te_s[r], 0, 0)),
                pl.BlockSpec(memory_space=pl.ANY),
                pl.BlockSpec((1, 1, _H),
                             lambda r, te_s, re_s, nx_s, nx2_s: (te_s[r], 0, 0)),
            ],
            out_specs=pl.BlockSpec((_G, _H), lambda r, te_s, re_s, nx_s, nx2_s: (r, 0)),
            scratch_shapes=[
                pltpu.VMEM((3, _H, _F), jnp.float32),
                pltpu.VMEM((3, _F, _H), jnp.float32),
                pltpu.SemaphoreType.DMA((3, 2)),
                pltpu.SMEM((2,), jnp.int32),
            ],
        ),
        out_shape=jax.ShapeDtypeStruct((_M, _H), jnp.float32),
        compiler_params=pltpu.CompilerParams(
            dimension_semantics=("arbitrary",),
        ),
    )(te.reshape(_NR), rowend.reshape(1), nxt.reshape(_NR),
      nxt2.reshape(_NR), xs, up_w, up_b3, down_w, down_b3)

    final = pl.kernel(
        _combine_body,
        out_type=jax.ShapeDtypeStruct((_T, _H), jnp.float32),
        mesh=plsc.VectorSubcoreMesh(core_axis_name="c", subcore_axis_name="s"),
        scratch_types=[
            pltpu.VMEM((_TPW // _CC, _CC), jnp.int32),
            pltpu.VMEM((_TPW // _CC, _CC), jnp.int32),
            pltpu.VMEM((_TPW // _CC, _CC, _H), jnp.float32),
            pltpu.VMEM((_TPW // _CC, _CC, _H), jnp.float32),
            pltpu.SemaphoreType.DMA((_TPW // _CC,)),
            pltpu.SemaphoreType.DMA((_TPW // _CC,)),
            pltpu.SemaphoreType.DMA((_TPW // _CC,)),
        ],
    )(ys, d1.reshape(_NW, _TPW // _CC, _CC), d2.reshape(_NW, _TPW // _CC, _CC))

    return final.reshape(b, s_len, d), logits


# speedup vs baseline: 1.0259x; 1.0259x over previous
import functools

import jax
import jax.numpy as jnp
from jax import lax
from jax.experimental import pallas as pl
from jax.experimental.pallas import tpu as pltpu
from jax.experimental.pallas import tpu_sc as plsc

_H = 768
_F = 1536
_E = 8
_T = 2048
_TT = 512
_NT = _T // _TT
_G = 128
_M = 2 * _T + _E * _G
_NR = _M // _G
_NC = 2
_NS = 16
_NW = _NC * _NS
_PAIRS = 2 * _T
_PPW = _PAIRS // _NW
_TPW = _T // _NW


def _route_body(x_ref, rw_ref, logits_ref, d1_ref, d2_ref, te_ref, re_ref,
                nxt_ref, nxt2_ref,
                logits_s, rank_s, idx1_s, idx2_s, cnt_s, tri_s):
    i = pl.program_id(0)

    @pl.when(i < _NT)
    def _():
        sl = pl.ds(i * _TT, _TT)

        @pl.when(i == 0)
        def _():
            cnt_s[...] = jnp.zeros_like(cnt_s)
            ri = lax.broadcasted_iota(jnp.int32, (_TT, _TT), 0)
            ci = lax.broadcasted_iota(jnp.int32, (_TT, _TT), 1)
            tri_s[...] = (ci < ri).astype(jnp.float32)

        x = x_ref[...]
        logits = lax.dot_general(
            x, rw_ref[...], (((1,), (0,)), ((), ())),
            precision=lax.Precision.DEFAULT,
            preferred_element_type=jnp.float32,
        )
        logits_s[sl, :] = logits
        lane = lax.broadcasted_iota(jnp.int32, (_TT, _E), 1)
        m1 = jnp.max(logits, axis=1, keepdims=True)
        idx1 = jnp.min(jnp.where(logits == m1, lane, _E), axis=1,
                       keepdims=True)
        neg = jnp.full_like(logits, -jnp.inf)
        l2 = jnp.where(lane == idx1, neg, logits)
        m2 = jnp.max(l2, axis=1, keepdims=True)
        idx2 = jnp.min(jnp.where(l2 == m2, lane, _E), axis=1, keepdims=True)
        idx1_s[sl, :] = idx1
        idx2_s[sl, :] = idx2
        mask = ((lane == idx1) | (lane == idx2)).astype(jnp.float32)
        rank = lax.dot_general(
            tri_s[...], mask, (((1,), (0,)), ((), ())),
            precision=lax.Precision.DEFAULT,
            preferred_element_type=jnp.float32,
        )
        rank_s[sl, :] = rank + cnt_s[...]
        cnt_s[...] = cnt_s[...] + jnp.sum(mask, axis=0, keepdims=True)

    @pl.when(i == _NT)
    def _():
        logits_ref[...] = logits_s[...]
        counts = cnt_s[...]
        pc = jnp.floor((counts + (_G - 1)) * (1.0 / _G)) * _G
        er = lax.broadcasted_iota(jnp.int32, (_E, _E), 0)
        ec = lax.broadcasted_iota(jnp.int32, (_E, _E), 1)
        ltri = (er < ec).astype(jnp.float32)
        poff = lax.dot_general(
            pc, ltri, (((1,), (0,)), ((), ())),
            precision=lax.Precision.DEFAULT,
            preferred_element_type=jnp.float32,
        )
        rank = rank_s[...]
        idx1 = idx1_s[...]
        idx2 = idx2_s[...]
        lane = lax.broadcasted_iota(jnp.int32, (_T, _E), 1)
        val = poff + rank
        d1 = jnp.sum(jnp.where(lane == idx1, val, 0.0), axis=1, keepdims=True)
        d2 = jnp.sum(jnp.where(lane == idx2, val, 0.0), axis=1, keepdims=True)
        d1_ref[...] = d1.astype(jnp.int32)
        d2_ref[...] = d2.astype(jnp.int32)
        ends = poff + pc
        rows = (lax.broadcasted_iota(jnp.int32, (_NR, _E), 0) * _G
                ).astype(jnp.float32)
        te = jnp.sum((rows >= ends).astype(jnp.int32), axis=1, keepdims=True)
        lane_g = lax.broadcasted_iota(jnp.int32, (1, _E), 1)
        gmax = jnp.max(jnp.where(pc > 0.0, lane_g, 0), axis=1, keepdims=True)
        te = jnp.minimum(te, gmax)
        te_ref[...] = te
        re_ref[...] = jnp.sum(pc, axis=1, keepdims=True).astype(jnp.int32)
        lane_r = lax.broadcasted_iota(jnp.int32, (_NR, _E), 1)
        present = pc > 0.0
        cand = jnp.where((lane_r > te) & present, lane_r, _E)
        nxt = jnp.min(cand, axis=1, keepdims=True)
        nxt = jnp.where(nxt == _E, te, nxt)
        nxt_ref[...] = nxt
        cand2 = jnp.where((lane_r > nxt) & present, lane_r, _E)
        nxt2 = jnp.min(cand2, axis=1, keepdims=True)
        nxt2_ref[...] = jnp.where(nxt2 == _E, nxt, nxt2)


_DCH = 4


def _dispatch_body(flat_hbm, dest_hbm, xs_hbm, dest_v, rbufs, gsems, ssems):
    c = lax.axis_index("c")
    s = lax.axis_index("s")
    wid = s * _NC + c
    ck = _PPW // _DCH
    pltpu.sync_copy(dest_hbm.at[wid], dest_v)
    tbase = lax.rem(wid, _NS) * _PPW
    gets = []
    for ch in range(_DCH):
        gets.append(pltpu.async_copy(
            flat_hbm.at[pl.ds(tbase + ch * ck, ck)], rbufs.at[ch],
            gsems.at[ch]))
    puts = []
    for ch in range(_DCH):
        gets[ch].wait()
        puts.append(pltpu.async_copy(
            rbufs.at[ch], xs_hbm.at[dest_v.at[ch]], ssems.at[ch]))
    for ch in range(_DCH):
        puts[ch].wait()


def _gmm_body(te_ref, re_ref, nx_ref, nx2_ref, xs_ref, upw_hbm, upb_ref,
              dww_hbm, dwb_ref,
              ys_ref, upw_b, dww_b, sems, ns_ref):
    r = pl.program_id(0)
    e = te_ref[r]
    prev_e = jnp.where(r > 0, te_ref[jnp.maximum(r - 1, 0)], -1)
    switch = e != prev_e
    ne = nx_ref[r]

    ne2 = nx2_ref[r]

    @pl.when(r == 0)
    def _():
        ns_ref[0] = 0
        ns_ref[1] = 0
        pltpu.make_async_copy(upw_hbm.at[e], upw_b.at[0],
                              sems.at[0, 0]).start()
        pltpu.make_async_copy(dww_hbm.at[e], dww_b.at[0],
                              sems.at[0, 1]).start()
        ns_ref[0] = 1

        @pl.when(ne != e)
        def _():
            pltpu.make_async_copy(upw_hbm.at[ne], upw_b.at[1],
                                  sems.at[1, 0]).start()
            pltpu.make_async_copy(dww_hbm.at[ne], dww_b.at[1],
                                  sems.at[1, 1]).start()
            ns_ref[0] = 2

    @pl.when(switch)
    def _():
        wslot = lax.rem(ns_ref[1], 3)
        for _k in range(3):
            @pl.when(wslot == _k)
            def _(_k=_k):
                pltpu.make_async_copy(upw_hbm.at[e], upw_b.at[_k],
                                      sems.at[_k, 0]).wait()
                pltpu.make_async_copy(dww_hbm.at[e], dww_b.at[_k],
                                      sems.at[_k, 1]).wait()
        ns_ref[1] = ns_ref[1] + 1

        @pl.when(ne2 != ne)
        def _():
            islot = lax.rem(ns_ref[0], 3)
            for _k in range(3):
                @pl.when(islot == _k)
                def _(_k=_k):
                    pltpu.make_async_copy(upw_hbm.at[ne2], upw_b.at[_k],
                                          sems.at[_k, 0]).start()
                    pltpu.make_async_copy(dww_hbm.at[ne2], dww_b.at[_k],
                                          sems.at[_k, 1]).start()
            ns_ref[0] = ns_ref[0] + 1

    slot = lax.rem(ns_ref[1] - 1, 3)
    active = r * _G < re_ref[0]

    def _compute(k):
        up = lax.dot_general(
            xs_ref[...], upw_b[k], (((1,), (0,)), ((), ())),
            precision=lax.Precision.DEFAULT,
            preferred_element_type=jnp.float32,
        )
        up = up + upb_ref[0]
        act = up * jax.nn.sigmoid(up)
        down = lax.dot_general(
            act, dww_b[k], (((1,), (0,)), ((), ())),
            precision=lax.Precision.DEFAULT,
            preferred_element_type=jnp.float32,
        )
        ys_ref[...] = down + dwb_ref[0]

    for _k in range(3):
        @pl.when(active & (slot == _k))
        def _(_k=_k):
            _compute(_k)


_CC = 32


def _combine_body(ys_hbm, d1_hbm, d2_hbm, out_hbm, i1, i2, r1, r2,
                  sem1, sem2):
    c = lax.axis_index("c")
    s = lax.axis_index("s")
    wid = s * _NC + c
    pltpu.sync_copy(d1_hbm.at[wid], i1)
    pltpu.sync_copy(d2_hbm.at[wid], i2)
    for ch in range(_TPW // _CC):
        g1 = pltpu.async_copy(ys_hbm.at[i1.at[ch]], r1, sem1)
        g2 = pltpu.async_copy(ys_hbm.at[i2.at[ch]], r2, sem2)
        g1.wait()
        g2.wait()

        def _row(row, carry):
            for col in range(_H // 16):
                sl = (row, pl.ds(col * 16, 16))
                r1[sl] = r1[sl] + r2[sl]
            return carry

        lax.fori_loop(0, _CC, _row, 0)
        pltpu.sync_copy(
            r1, out_hbm.at[pl.ds(wid * _TPW + ch * _CC, _CC)])


@functools.partial(jax.jit, static_argnames=())
def kernel(hidden_states, router_w, up_w, up_b, down_w, down_b):
    b, s_len, d = hidden_states.shape
    flat = hidden_states.reshape(_T, d)

    logits, d1, d2, te, rowend, nxt, nxt2 = pl.pallas_call(
        _route_body,
        grid=(_NT + 1,),
        in_specs=[
            pl.BlockSpec((_TT, _H), lambda i: (jnp.minimum(i, _NT - 1), 0)),
            pl.BlockSpec((_H, _E), lambda i: (0, 0)),
        ],
        out_specs=(
            pl.BlockSpec((_T, _E), lambda i: (0, 0)),
            pl.BlockSpec((_T, 1), lambda i: (0, 0)),
            pl.BlockSpec((_T, 1), lambda i: (0, 0)),
            pl.BlockSpec((_NR, 1), lambda i: (0, 0)),
            pl.BlockSpec((1, 1), lambda i: (0, 0)),
            pl.BlockSpec((_NR, 1), lambda i: (0, 0)),
            pl.BlockSpec((_NR, 1), lambda i: (0, 0)),
        ),
        out_shape=(
            jax.ShapeDtypeStruct((_T, _E), jnp.float32),
            jax.ShapeDtypeStruct((_T, 1), jnp.int32),
            jax.ShapeDtypeStruct((_T, 1), jnp.int32),
            jax.ShapeDtypeStruct((_NR, 1), jnp.int32),
            jax.ShapeDtypeStruct((1, 1), jnp.int32),
            jax.ShapeDtypeStruct((_NR, 1), jnp.int32),
            jax.ShapeDtypeStruct((_NR, 1), jnp.int32),
        ),
        scratch_shapes=[
            pltpu.VMEM((_T, _E), jnp.float32),
            pltpu.VMEM((_T, _E), jnp.float32),
            pltpu.VMEM((_T, 1), jnp.int32),
            pltpu.VMEM((_T, 1), jnp.int32),
            pltpu.VMEM((1, _E), jnp.float32),
            pltpu.VMEM((_TT, _TT), jnp.float32),
        ],
        compiler_params=pltpu.CompilerParams(
            dimension_semantics=("arbitrary",),
        ),
    )(flat, router_w)

    dest = jnp.concatenate([d1.reshape(_T), d2.reshape(_T)])
    dest3 = dest.reshape(_NW, _DCH, _PPW // _DCH)

    mesh = plsc.VectorSubcoreMesh(core_axis_name="c", subcore_axis_name="s")
    xs = pl.kernel(
        _dispatch_body,
        out_type=jax.ShapeDtypeStruct((_M, _H), jnp.float32),
        mesh=mesh,
        scratch_types=[
            pltpu.VMEM((_DCH, _PPW // _DCH), jnp.int32),
            pltpu.VMEM((_DCH, _PPW // _DCH, _H), jnp.float32),
            pltpu.SemaphoreType.DMA((_DCH,)),
            pltpu.SemaphoreType.DMA((_DCH,)),
        ],
    )(flat, dest3)

    up_b3 = up_b.reshape(_E, 1, _F)
    down_b3 = down_b.reshape(_E, 1, _H)
    ys = pl.pallas_call(
        _gmm_body,
        grid_spec=pltpu.PrefetchScalarGridSpec(
            num_scalar_prefetch=4,
            grid=(_NR,),
            in_specs=[
                pl.BlockSpec((_G, _H), lambda r, te_s, re_s, nx_s, nx2_s: (r, 0)),
                pl.BlockSpec(memory_space=pl.ANY),
                pl.BlockSpec((1, 1, _F),
                             lambda r, te_s, re_s, nx_s, nx2_s: (te_s[r], 0, 0)),
                pl.BlockSpec(memory_space=pl.ANY),
                pl.BlockSpec((1, 1, _H),
                             lambda r, te_s, re_s, nx_s, nx2_s: (te_s[r], 0, 0)),
            ],
            out_specs=pl.BlockSpec((_G, _H), lambda r, te_s, re_s, nx_s, nx2_s: (r, 0)),
            scratch_shapes=[
                pltpu.VMEM((3, _H, _F), jnp.float32),
                pltpu.VMEM((3, _F, _H), jnp.float32),
                pltpu.SemaphoreType.DMA((3, 2)),
                pltpu.SMEM((2,), jnp.int32),
            ],
        ),
        out_shape=jax.ShapeDtypeStruct((_M, _H), jnp.float32),
        compiler_params=pltpu.CompilerParams(
            dimension_semantics=("arbitrary",),
        ),
    )(te.reshape(_NR), rowend.reshape(1), nxt.reshape(_NR),
      nxt2.reshape(_NR), xs, up_w, up_b3, down_w, down_b3)

    final = pl.kernel(
        _combine_body,
        out_type=jax.ShapeDtypeStruct((_T, _H), jnp.float32),
        mesh=plsc.VectorSubcoreMesh(core_axis_name="c", subcore_axis_name="s"),
        scratch_types=[
            pltpu.VMEM((_TPW // _CC, _CC), jnp.int32),
            pltpu.VMEM((_TPW // _CC, _CC), jnp.int32),
            pltpu.VMEM((_CC, _H), jnp.float32),
            pltpu.VMEM((_CC, _H), jnp.float32),
            pltpu.SemaphoreType.DMA,
            pltpu.SemaphoreType.DMA,
        ],
    )(ys, d1.reshape(_NW, _TPW // _CC, _CC), d2.reshape(_NW, _TPW // _CC, _CC))

    return final.reshape(b, s_len, d), logits
